# baseline (speedup 1.0000x reference)
import functools

import jax
import jax.numpy as jnp
from jax import lax
from jax.experimental import pallas as pl
from jax.experimental.pallas import tpu as pltpu
from jax.experimental.pallas import tpu_sc as plsc

DIM = 32
NE = 8192
NROWS = 16384
R = 512
CB = 2048
RB = 256

NC = 2
NS = 16
NW = NC * NS
BPW = NROWS // NW


def _dist_kernel(z_ref, zsq_ref, e_ref, esq_ref, idx_ref, mv_ref):
    zb = z_ref[...].astype(jnp.float32)
    zsq = zsq_ref[...][:, None]

    def body(c, carry):
        run_m, run_i = carry
        et = e_ref[pl.ds(c * CB, CB), :]
        esq = esq_ref[pl.ds(c * CB, CB)]
        dot = lax.dot_general(zb, et, (((1,), (1,)), ((), ())),
                              precision=jax.lax.Precision.HIGHEST,
                              preferred_element_type=jnp.float32)
        d = (zsq + esq[None, :]) - 2.0 * dot
        m = jnp.min(d, axis=1)
        ii = jnp.min(
            jnp.where(d == m[:, None],
                      lax.broadcasted_iota(jnp.int32, (R, CB), 1),
                      jnp.int32(2 ** 30)),
            axis=1) + c * CB
        take = m < run_m
        return (jnp.where(take, m, run_m), jnp.where(take, ii, run_i))

    init = (jnp.full((R,), jnp.inf, jnp.float32), jnp.zeros((R,), jnp.int32))
    run_m, run_i = lax.fori_loop(0, NE // CB, body, init)
    idx_ref[...] = run_i
    mv_ref[...] = run_m


def _fin_kernel(idx_ref, zp_ref, zqw_ref, zqst_ref, hist_ref, loss_ref, perp_ref):
    i = pl.program_id(0)
    nb = pl.num_programs(0)
    zp = zp_ref[...]
    w = zqw_ref[...]
    c = (idx_ref[...] & 3)[:, None]
    zq = jnp.where(
        c == 0, w[:, 0:DIM],
        jnp.where(c == 1, w[:, DIM:2 * DIM],
                  jnp.where(c == 2, w[:, 2 * DIM:3 * DIM], w[:, 3 * DIM:])))
    zq = zq.astype(jnp.bfloat16).astype(jnp.float32)
    zqst_ref[...] = zp + (zq - zp)
    dif = zq - zp
    s = jnp.sum(dif * dif)
    idxb = idx_ref[...]
    onehot = jnp.where(
        idxb[:, None] == lax.broadcasted_iota(jnp.int32, (RB, NE), 1),
        1.0, 0.0)
    hsum = jnp.sum(onehot, axis=0)

    @pl.when(i == 0)
    def _init():
        hist_ref[...] = jnp.zeros((NE,), jnp.float32)
        loss_ref[...] = jnp.zeros((1, 1), jnp.float32)
        perp_ref[...] = jnp.zeros((1, 1), jnp.float32)

    hist_ref[...] += hsum
    loss_ref[...] = loss_ref[...] + s

    @pl.when(i == nb - 1)
    def _fin():
        e_mean = hist_ref[...] * (1.0 / NROWS)
        ent = jnp.sum(e_mean * jnp.log(e_mean + 1e-10))
        perp_ref[...] = jnp.broadcast_to(jnp.exp(-ent), (1, 1))
        m = loss_ref[...] / (NROWS * DIM)
        loss_ref[...] = m + 0.25 * m


@functools.cache
def _make_sc_gather():
    mesh = plsc.VectorSubcoreMesh(core_axis_name="c", subcore_axis_name="s")

    @functools.partial(
        pl.kernel,
        mesh=mesh,
        out_type=jax.ShapeDtypeStruct((NROWS, 4 * DIM), jnp.float32),
        scratch_types=[
            pltpu.VMEM((BPW // 128, 128), jnp.int32),
            pltpu.VMEM((BPW // 128, 128), jnp.int32),
            pltpu.VMEM((BPW, 4 * DIM), jnp.float32),
            pltpu.SemaphoreType.DMA,
        ],
    )
    def _sc_gather(idx_hbm, table_hbm, out_hbm, idx_v, idx4_v, rows_v, sem):
        wid = lax.axis_index("s") * NC + lax.axis_index("c")
        nrow = BPW // 128
        pltpu.sync_copy(idx_hbm.at[pl.ds(wid * nrow, nrow)], idx_v)
        for j in range(nrow):
            for k in range(128 // 16):
                v = idx_v[j, pl.ds(k * 16, 16)]
                idx4_v[j, pl.ds(k * 16, 16)] = v >> 2
        for j in range(nrow):
            pltpu.async_copy(table_hbm.at[idx4_v.at[j]],
                             rows_v.at[pl.ds(j * 128, 128)], sem).wait()
        pltpu.sync_copy(rows_v, out_hbm.at[pl.ds(wid * BPW, BPW)])

    return _sc_gather


def kernel(z, embed_weight):
    zp = jnp.transpose(z, (0, 2, 3, 1)).reshape(NROWS, DIM)

    zsq = jnp.sum(zp ** 2, axis=1)
    esq = jnp.sum(embed_weight ** 2, axis=1)
    idx, mv = pl.pallas_call(
        _dist_kernel,
        grid=(NROWS // R,),
        in_specs=[
            pl.BlockSpec((R, DIM), lambda i: (i, 0)),
            pl.BlockSpec((R,), lambda i: (i,)),
            pl.BlockSpec((NE, DIM), lambda i: (0, 0)),
            pl.BlockSpec((NE,), lambda i: (0,)),
        ],
        out_specs=[
            pl.BlockSpec((R,), lambda i: (i,)),
            pl.BlockSpec((R,), lambda i: (i,)),
        ],
        out_shape=[
            jax.ShapeDtypeStruct((NROWS,), jnp.int32),
            jax.ShapeDtypeStruct((NROWS,), jnp.float32),
        ],
    )(zp.astype(jnp.bfloat16), zsq, embed_weight, esq)

    zqw = _make_sc_gather()(idx.reshape(NROWS // 128, 128),
                            embed_weight.reshape(NE // 4, 4 * DIM))

    zqst, hist, loss11, perp11 = pl.pallas_call(
        _fin_kernel,
        grid=(NROWS // RB,),
        in_specs=[
            pl.BlockSpec((RB,), lambda i: (i,)),
            pl.BlockSpec((RB, DIM), lambda i: (i, 0)),
            pl.BlockSpec((RB, 4 * DIM), lambda i: (i, 0)),
        ],
        out_specs=[
            pl.BlockSpec((RB, DIM), lambda i: (i, 0)),
            pl.BlockSpec((NE,), lambda i: (0,)),
            pl.BlockSpec((1, 1), lambda i: (0, 0)),
            pl.BlockSpec((1, 1), lambda i: (0, 0)),
        ],
        out_shape=[
            jax.ShapeDtypeStruct((NROWS, DIM), jnp.float32),
            jax.ShapeDtypeStruct((NE,), jnp.float32),
            jax.ShapeDtypeStruct((1, 1), jnp.float32),
            jax.ShapeDtypeStruct((1, 1), jnp.float32),
        ],
    )(idx, zp, zqw)

    del hist
    z_q_out = zqst.reshape(16, 32, 32, DIM).transpose(0, 3, 1, 2)
    return loss11[0, 0], z_q_out, idx[:, None], perp11[0, 0]
